# baseline (speedup 1.0000x reference)
import functools

import jax
import jax.numpy as jnp
from jax import lax
from jax.experimental import pallas as pl
from jax.experimental.pallas import tpu as pltpu
from jax.experimental.pallas import tpu_sc as plsc

_NUM_CORES = 2
_NUM_SUBCORES = 16
_NUM_WORKERS = _NUM_CORES * _NUM_SUBCORES
_LANES = 16

_BATCH = 16384
_SEQ = 50
_DIM = 32
_OUT_DIM = 16

_ROWS_W = _BATCH // _NUM_WORKERS
_CB = 16
_G = _CB * _SEQ
_SUB = 80
_NSUB = _G // _SUB
_NCHUNK = _ROWS_W // _CB
_VOCAB = 1000000

_PROJ_BLK = 65536

_mesh = plsc.VectorSubcoreMesh(core_axis_name="c", subcore_axis_name="s")


@functools.partial(
    pl.kernel,
    out_type=jax.ShapeDtypeStruct((_BATCH, _OUT_DIM), jnp.float32),
    mesh=_mesh,
    scratch_types=[
        pltpu.VMEM((2, _NSUB, _SUB), jnp.int32),
        pltpu.VMEM((2, _G, _OUT_DIM), jnp.float32),
        pltpu.VMEM((_ROWS_W, _OUT_DIM), jnp.float32),
        pltpu.SemaphoreType.DMA,
        pltpu.SemaphoreType.DMA,
    ],
    compiler_params=pltpu.CompilerParams(use_tc_tiling_on_sc=False),
)
def _gather_pool(idx_hbm, table_hbm, out_hbm, idx_v, rows_v, out_v, sem0, sem1):
    wid = lax.axis_index("s") * _NUM_CORES + lax.axis_index("c")
    ibase = wid * _NCHUNK
    sems = (sem0, sem1)

    def fire(chunk, b):
        pltpu.sync_copy(idx_hbm.at[ibase + chunk], idx_v.at[b])
        for j in range(_NSUB):
            pltpu.async_copy(
                table_hbm.at[idx_v.at[b].at[j]],
                rows_v.at[b].at[pl.ds(j * _SUB, _SUB)],
                sems[b],
            )

    def drain(b):
        pltpu.make_async_copy(
            table_hbm.at[pl.ds(0, _G)], rows_v.at[b], sems[b]
        ).wait()

    def reduce(chunk, b):
        rows = rows_v.at[b]

        @pl.loop(0, _CB)
        def _row(r):
            base = r * _SEQ
            acc = [None] * 4
            for t in range(_SEQ):
                v = rows[base + t]
                a = t % 4
                acc[a] = v if acc[a] is None else acc[a] + v
            out_v[chunk * _CB + r] = (acc[0] + acc[1]) + (acc[2] + acc[3])

    fire(0, 0)

    @pl.loop(0, _NCHUNK - 2, step=2)
    def _chunks(c):
        for b in range(2):
            fire(c + b + 1, 1 - b)
            drain(b)
            reduce(c + b, b)

    fire(_NCHUNK - 1, 1)
    drain(0)
    reduce(_NCHUNK - 2, 0)
    drain(1)
    reduce(_NCHUNK - 1, 1)

    pltpu.sync_copy(out_v, out_hbm.at[pl.ds(wid * _ROWS_W, _ROWS_W)])


def _project_table(emb_t, projection):
    pack = 128 // _OUT_DIM

    def body(a_ref, p_ref, o_ref, d_ref):
        d_ref[...] = lax.dot_general(
            a_ref[...],
            p_ref[...],
            (((0,), (0,)), ((), ())),
            preferred_element_type=jnp.float32,
        )
        lane = lax.broadcasted_iota(jnp.int32, (_PROJ_BLK // pack, 128), 1)
        group = lane // _OUT_DIM
        acc = d_ref[0::pack, :]
        for m in range(1, pack):
            acc = jnp.where(group == m, d_ref[m::pack, :], acc)
        o_ref[...] = acc

    grid = (_VOCAB + _PROJ_BLK - 1) // _PROJ_BLK
    p_tiled = jnp.tile(projection * (1.0 / _SEQ), (1, pack))
    out = pl.pallas_call(
        body,
        grid=(grid,),
        in_specs=[
            pl.BlockSpec((_DIM, _PROJ_BLK), lambda i: (0, i)),
            pl.BlockSpec((_DIM, 128), lambda i: (0, 0)),
        ],
        out_specs=pl.BlockSpec((_PROJ_BLK // pack, 128), lambda i: (i, 0)),
        out_shape=jax.ShapeDtypeStruct((_VOCAB // pack, 128), jnp.float32),
        scratch_shapes=[pltpu.VMEM((_PROJ_BLK, 128), jnp.float32)],
        compiler_params=pltpu.CompilerParams(fuse_transposed_lhs_in_matmul=True),
    )(emb_t, p_tiled)
    return out.reshape(_VOCAB, _OUT_DIM)


def kernel(token_ids, embeddings, projection):
    flat = token_ids.astype(jnp.int32).reshape(_NUM_WORKERS * _NCHUNK, _NSUB, _SUB)
    table_p = _project_table(embeddings.T, projection)
    return _gather_pool(flat, table_p)


# speedup vs baseline: 1.0105x; 1.0105x over previous
import functools

import jax
import jax.numpy as jnp
from jax import lax
from jax.experimental import pallas as pl
from jax.experimental.pallas import tpu as pltpu
from jax.experimental.pallas import tpu_sc as plsc

_NUM_CORES = 2
_NUM_SUBCORES = 16
_NUM_WORKERS = _NUM_CORES * _NUM_SUBCORES
_LANES = 16

_BATCH = 16384
_SEQ = 50
_DIM = 32
_OUT_DIM = 16

_ROWS_W = _BATCH // _NUM_WORKERS
_CB = 16
_G = _CB * _SEQ
_SUB = 80
_NSUB = _G // _SUB
_NCHUNK = _ROWS_W // _CB
_VOCAB = 1000000

_PROJ_BLK = 32768

_mesh = plsc.VectorSubcoreMesh(core_axis_name="c", subcore_axis_name="s")


@functools.partial(
    pl.kernel,
    out_type=jax.ShapeDtypeStruct((_BATCH, _OUT_DIM), jnp.float32),
    mesh=_mesh,
    scratch_types=[
        pltpu.VMEM((2, _NSUB, _SUB), jnp.int32),
        pltpu.VMEM((2, _G, _OUT_DIM), jnp.float32),
        pltpu.VMEM((_ROWS_W, _OUT_DIM), jnp.float32),
        pltpu.SemaphoreType.DMA,
        pltpu.SemaphoreType.DMA,
    ],
    compiler_params=pltpu.CompilerParams(use_tc_tiling_on_sc=False),
)
def _gather_pool(idx_hbm, table_hbm, out_hbm, idx_v, rows_v, out_v, sem0, sem1):
    wid = lax.axis_index("s") * _NUM_CORES + lax.axis_index("c")
    ibase = wid * _NCHUNK
    sems = (sem0, sem1)

    def fire(chunk, b):
        pltpu.sync_copy(idx_hbm.at[ibase + chunk], idx_v.at[b])
        for j in range(_NSUB):
            pltpu.async_copy(
                table_hbm.at[idx_v.at[b].at[j]],
                rows_v.at[b].at[pl.ds(j * _SUB, _SUB)],
                sems[b],
            )

    def drain(b):
        pltpu.make_async_copy(
            table_hbm.at[pl.ds(0, _G)], rows_v.at[b], sems[b]
        ).wait()

    def reduce(chunk, b):
        rows = rows_v.at[b]

        @pl.loop(0, _CB)
        def _row(r):
            base = r * _SEQ
            acc = [None] * 4
            for t in range(_SEQ):
                v = rows[base + t]
                a = t % 4
                acc[a] = v if acc[a] is None else acc[a] + v
            out_v[chunk * _CB + r] = (acc[0] + acc[1]) + (acc[2] + acc[3])

    fire(0, 0)

    @pl.loop(0, _NCHUNK - 2, step=2)
    def _chunks(c):
        for b in range(2):
            fire(c + b + 1, 1 - b)
            drain(b)
            reduce(c + b, b)

    fire(_NCHUNK - 1, 1)
    drain(0)
    reduce(_NCHUNK - 2, 0)
    drain(1)
    reduce(_NCHUNK - 1, 1)

    pltpu.sync_copy(out_v, out_hbm.at[pl.ds(wid * _ROWS_W, _ROWS_W)])


def _project_table(emb_t, projection):
    pack = 128 // _OUT_DIM

    def body(a_ref, p_ref, o_ref, d_ref):
        d_ref[...] = lax.dot_general(
            a_ref[...],
            p_ref[...],
            (((0,), (0,)), ((), ())),
            preferred_element_type=jnp.float32,
        )
        lane = lax.broadcasted_iota(jnp.int32, (_PROJ_BLK // pack, 128), 1)
        group = lane // _OUT_DIM
        acc = d_ref[0::pack, :]
        for m in range(1, pack):
            acc = jnp.where(group == m, d_ref[m::pack, :], acc)
        o_ref[...] = acc

    grid = (_VOCAB + _PROJ_BLK - 1) // _PROJ_BLK
    p_tiled = jnp.tile(projection * (1.0 / _SEQ), (1, pack))
    out = pl.pallas_call(
        body,
        grid=(grid,),
        in_specs=[
            pl.BlockSpec((_DIM, _PROJ_BLK), lambda i: (0, i)),
            pl.BlockSpec((_DIM, 128), lambda i: (0, 0)),
        ],
        out_specs=pl.BlockSpec((_PROJ_BLK // pack, 128), lambda i: (i, 0)),
        out_shape=jax.ShapeDtypeStruct((_VOCAB // pack, 128), jnp.float32),
        scratch_shapes=[pltpu.VMEM((_PROJ_BLK, 128), jnp.float32)],
        compiler_params=pltpu.CompilerParams(fuse_transposed_lhs_in_matmul=True),
    )(emb_t, p_tiled)
    return out.reshape(_VOCAB, _OUT_DIM)


def kernel(token_ids, embeddings, projection):
    flat = token_ids.astype(jnp.int32).reshape(_NUM_WORKERS * _NCHUNK, _NSUB, _SUB)
    table_p = _project_table(embeddings.T, projection)
    return _gather_pool(flat, table_p)


# speedup vs baseline: 1.0385x; 1.0276x over previous
import functools

import jax
import jax.numpy as jnp
from jax import lax
from jax.experimental import pallas as pl
from jax.experimental.pallas import tpu as pltpu
from jax.experimental.pallas import tpu_sc as plsc

_NUM_CORES = 2
_NUM_SUBCORES = 16
_NUM_WORKERS = _NUM_CORES * _NUM_SUBCORES
_LANES = 16

_BATCH = 16384
_SEQ = 50
_DIM = 32
_OUT_DIM = 16

_ROWS_W = _BATCH // _NUM_WORKERS
_CB = 32
_G = _CB * _SEQ
_SUB = 80
_NSUB = _G // _SUB
_NCHUNK = _ROWS_W // _CB
_NROWIDX = _ROWS_W * _SEQ // _SUB
_VOCAB = 1000000

_PROJ_BLK = 32768

_mesh = plsc.VectorSubcoreMesh(core_axis_name="c", subcore_axis_name="s")


@functools.partial(
    pl.kernel,
    out_type=jax.ShapeDtypeStruct((_BATCH, _OUT_DIM), jnp.float32),
    mesh=_mesh,
    scratch_types=[
        pltpu.VMEM((_NROWIDX, _SUB), jnp.int32),
        pltpu.VMEM((2, _G, _OUT_DIM), jnp.float32),
        pltpu.VMEM((_ROWS_W, _OUT_DIM), jnp.float32),
        pltpu.SemaphoreType.DMA,
        pltpu.SemaphoreType.DMA,
    ],
    compiler_params=pltpu.CompilerParams(use_tc_tiling_on_sc=False),
)
def _gather_pool(idx_hbm, table_hbm, out_hbm, idx_v, rows_v, out_v, sem0, sem1):
    wid = lax.axis_index("s") * _NUM_CORES + lax.axis_index("c")
    sems = (sem0, sem1)

    pltpu.sync_copy(idx_hbm.at[wid], idx_v)

    def fire(chunk, b):
        for j in range(_NSUB):
            pltpu.async_copy(
                table_hbm.at[idx_v.at[chunk * _NSUB + j]],
                rows_v.at[b].at[pl.ds(j * _SUB, _SUB)],
                sems[b],
            )

    def drain(b):
        pltpu.make_async_copy(
            table_hbm.at[pl.ds(0, _G)], rows_v.at[b], sems[b]
        ).wait()

    def reduce(chunk, b):
        rows = rows_v.at[b]

        @pl.loop(0, _CB)
        def _row(r):
            base = r * _SEQ
            acc = [None] * 4
            for t in range(_SEQ):
                v = rows[base + t]
                a = t % 4
                acc[a] = v if acc[a] is None else acc[a] + v
            out_v[chunk * _CB + r] = (acc[0] + acc[1]) + (acc[2] + acc[3])

    fire(0, 0)

    @pl.loop(0, _NCHUNK - 2, step=2)
    def _chunks(c):
        for b in range(2):
            fire(c + b + 1, 1 - b)
            drain(b)
            reduce(c + b, b)

    fire(_NCHUNK - 1, 1)
    drain(0)
    reduce(_NCHUNK - 2, 0)
    drain(1)
    reduce(_NCHUNK - 1, 1)

    pltpu.sync_copy(out_v, out_hbm.at[pl.ds(wid * _ROWS_W, _ROWS_W)])


def _project_table(emb_t, projection):
    pack = 128 // _OUT_DIM

    def body(a_ref, p_ref, o_ref, d_ref):
        d_ref[...] = lax.dot_general(
            a_ref[...],
            p_ref[...],
            (((0,), (0,)), ((), ())),
            preferred_element_type=jnp.float32,
        )
        lane = lax.broadcasted_iota(jnp.int32, (_PROJ_BLK // pack, 128), 1)
        group = lane // _OUT_DIM
        acc = d_ref[0::pack, :]
        for m in range(1, pack):
            acc = jnp.where(group == m, d_ref[m::pack, :], acc)
        o_ref[...] = acc

    grid = (_VOCAB + _PROJ_BLK - 1) // _PROJ_BLK
    p_tiled = jnp.tile(projection * (1.0 / _SEQ), (1, pack))
    out = pl.pallas_call(
        body,
        grid=(grid,),
        in_specs=[
            pl.BlockSpec((_DIM, _PROJ_BLK), lambda i: (0, i)),
            pl.BlockSpec((_DIM, 128), lambda i: (0, 0)),
        ],
        out_specs=pl.BlockSpec((_PROJ_BLK // pack, 128), lambda i: (i, 0)),
        out_shape=jax.ShapeDtypeStruct((_VOCAB // pack, 128), jnp.float32),
        scratch_shapes=[pltpu.VMEM((_PROJ_BLK, 128), jnp.float32)],
        compiler_params=pltpu.CompilerParams(fuse_transposed_lhs_in_matmul=True),
    )(emb_t, p_tiled)
    return out.reshape(_VOCAB, _OUT_DIM)


def kernel(token_ids, embeddings, projection):
    flat = token_ids.astype(jnp.int32).reshape(_NUM_WORKERS, _NROWIDX, _SUB)
    table_p = _project_table(embeddings.T, projection)
    return _gather_pool(flat, table_p)


# speedup vs baseline: 1.2839x; 1.2364x over previous
import functools

import jax
import jax.numpy as jnp
from jax import lax
from jax.experimental import pallas as pl
from jax.experimental.pallas import tpu as pltpu
from jax.experimental.pallas import tpu_sc as plsc

_NUM_CORES = 2
_NUM_SUBCORES = 16
_NUM_WORKERS = _NUM_CORES * _NUM_SUBCORES
_LANES = 16

_BATCH = 16384
_SEQ = 50
_DIM = 32
_OUT_DIM = 16

_ROWS_W = _BATCH // _NUM_WORKERS
_CB = 32
_G = _CB * _SEQ
_SUB = 80
_NSUB = _G // _SUB
_NCHUNK = _ROWS_W // _CB
_NROWIDX = _ROWS_W * _SEQ // _SUB
_VOCAB = 1000000

_PROJ_BLK = 32768

_mesh = plsc.VectorSubcoreMesh(core_axis_name="c", subcore_axis_name="s")


@functools.partial(
    pl.kernel,
    out_type=jax.ShapeDtypeStruct((_BATCH, _OUT_DIM), jnp.float32),
    mesh=_mesh,
    scratch_types=[
        pltpu.VMEM((_NROWIDX, _SUB), jnp.int32),
        pltpu.VMEM((2, _G, _OUT_DIM), jnp.float32),
        pltpu.VMEM((_ROWS_W, _OUT_DIM), jnp.float32),
        pltpu.SemaphoreType.DMA,
        pltpu.SemaphoreType.DMA,
    ],
    compiler_params=pltpu.CompilerParams(use_tc_tiling_on_sc=False),
)
def _gather_pool(idx_hbm, table_hbm, out_hbm, idx_v, rows_v, out_v, sem0, sem1):
    wid = lax.axis_index("s") * _NUM_CORES + lax.axis_index("c")
    sems = (sem0, sem1)

    pltpu.sync_copy(idx_hbm.at[wid], idx_v)

    def fire(chunk, b):
        for j in range(_NSUB):
            pltpu.async_copy(
                table_hbm.at[idx_v.at[chunk * _NSUB + j]],
                rows_v.at[b].at[pl.ds(j * _SUB, _SUB)],
                sems[b],
            )

    def drain(b):
        pltpu.make_async_copy(
            table_hbm.at[pl.ds(0, _G)], rows_v.at[b], sems[b]
        ).wait()

    def reduce(chunk, b):
        rows = rows_v.at[b]

        @pl.loop(0, _CB)
        def _row(r):
            base = r * _SEQ
            acc = [None] * 4
            for t in range(_SEQ):
                v = rows[base + t]
                a = t % 4
                acc[a] = v if acc[a] is None else acc[a] + v
            out_v[chunk * _CB + r] = (acc[0] + acc[1]) + (acc[2] + acc[3])

    fire(0, 0)

    @pl.loop(0, _NCHUNK - 2, step=2)
    def _chunks(c):
        for b in range(2):
            fire(c + b + 1, 1 - b)
            drain(b)
            reduce(c + b, b)

    fire(_NCHUNK - 1, 1)
    drain(0)
    reduce(_NCHUNK - 2, 0)
    drain(1)
    reduce(_NCHUNK - 1, 1)

    pltpu.sync_copy(out_v, out_hbm.at[pl.ds(wid * _ROWS_W, _ROWS_W)])


def _project_table(emb_t, projection):
    pack = 128 // _OUT_DIM

    def body(a_ref, p_ref, o_ref, d_ref):
        d_ref[...] = lax.dot_general(
            a_ref[...].astype(jnp.bfloat16),
            p_ref[...].astype(jnp.bfloat16),
            (((0,), (0,)), ((), ())),
            preferred_element_type=jnp.float32,
        )
        lane = lax.broadcasted_iota(jnp.int32, (_PROJ_BLK // pack, 128), 1)
        group = lane // _OUT_DIM
        acc = d_ref[0::pack, :]
        for m in range(1, pack):
            acc = jnp.where(group == m, d_ref[m::pack, :], acc)
        o_ref[...] = acc

    grid = (_VOCAB + _PROJ_BLK - 1) // _PROJ_BLK
    p_tiled = jnp.tile(projection * (1.0 / _SEQ), (1, pack))
    out = pl.pallas_call(
        body,
        grid=(grid,),
        in_specs=[
            pl.BlockSpec((_DIM, _PROJ_BLK), lambda i: (0, i)),
            pl.BlockSpec((_DIM, 128), lambda i: (0, 0)),
        ],
        out_specs=pl.BlockSpec((_PROJ_BLK // pack, 128), lambda i: (i, 0)),
        out_shape=jax.ShapeDtypeStruct((_VOCAB // pack, 128), jnp.float32),
        scratch_shapes=[pltpu.VMEM((_PROJ_BLK, 128), jnp.float32)],
    )(emb_t, p_tiled)
    return out.reshape(_VOCAB, _OUT_DIM)


def kernel(token_ids, embeddings, projection):
    flat = token_ids.astype(jnp.int32).reshape(_NUM_WORKERS, _NROWIDX, _SUB)
    table_p = _project_table(embeddings.T, projection)
    return _gather_pool(flat, table_p)
